# baseline (speedup 1.0000x reference)
import functools

import jax
import jax.numpy as jnp
from jax import lax
from jax.experimental import pallas as pl
from jax.experimental.pallas import tpu as pltpu
from jax.experimental.pallas import tpu_sc as plsc

_IDXW = 128
_SLABW = 133


@functools.lru_cache(maxsize=None)
def _build_transpose(V, D, NC, NS):
    NW = NC * NS
    RB = D // 8
    TU = 2 * _IDXW
    SW = TU + 5
    full_units = V // TU
    tail = V - full_units * TU
    units_per_w = full_units // NW
    extra = full_units - units_per_w * NW
    n_outer = units_per_w // 2
    odd = units_per_w - 2 * n_outer

    mesh = plsc.VectorSubcoreMesh(core_axis_name="c", subcore_axis_name="s")

    @functools.partial(
        pl.kernel,
        out_type=jax.ShapeDtypeStruct((V * D,), jnp.float32),
        mesh=mesh,
        scratch_types=[
            pltpu.VMEM((2, RB, 8, SW), jnp.float32),
            pltpu.VMEM((TU * D,), jnp.float32),
            pltpu.VMEM((max(tail, 1) * D,), jnp.float32),
            pltpu.SemaphoreType.DMA,
            pltpu.SemaphoreType.DMA,
        ],
        compiler_params=pltpu.CompilerParams(use_tc_tiling_on_sc=True,
                                             needs_layout_passes=False),
    )
    def transpose_kernel(tT_hbm, tail_hbm, out_hbm, slab_v, rows_v, tail_v,
                         sem_in, sem_st):
        wid = lax.axis_index("s") * NC + lax.axis_index("c")
        base = wid * units_per_w

        rv = lax.iota(jnp.int32, 16) & 7
        rb_lo = lax.iota(jnp.int32, 16) >> 3
        rb_hi = rb_lo + 2

        def fire_loads(u, s):
            tok0 = pl.multiple_of(u * TU, _IDXW)
            for rb in range(RB):
                pltpu.async_copy(
                    tT_hbm.at[pl.ds(rb * 8, 8), pl.ds(tok0, TU)],
                    slab_v.at[s, rb, pl.ds(0, 8), pl.ds(0, TU)], sem_in)

        def wait_loads(s):
            for rb in range(RB):
                pltpu.make_async_copy(
                    tT_hbm.at[pl.ds(0, 8), pl.ds(0, TU)],
                    slab_v.at[s, rb, pl.ds(0, 8), pl.ds(0, TU)],
                    sem_in).wait()

        def wait_store():
            pltpu.make_async_copy(rows_v, out_hbm.at[pl.ds(0, TU * D)],
                                  sem_st).wait()

        def transpose(s):
            @pl.loop(0, TU, unroll=8)
            def _tok(i):
                cv = jnp.full((16,), i, jnp.int32)
                lo = plsc.load_gather(slab_v.at[s], [rb_lo, rv, cv])
                hi = plsc.load_gather(slab_v.at[s], [rb_hi, rv, cv])
                rows_v[pl.ds(i * D, 16)] = lo
                rows_v[pl.ds(i * D + 16, 16)] = hi

        def store(u):
            pltpu.async_copy(rows_v, out_hbm.at[pl.ds(u * (TU * D), TU * D)],
                             sem_st)

        fire_loads(base, 0)

        @pl.loop(0, n_outer)
        def _outer(k):
            for s in range(2):
                j = k * 2 + s
                wait_loads(s)

                @pl.when(j < units_per_w - 1)
                def _():
                    fire_loads(base + j + 1, 1 - s)

                @pl.when(j > 0)
                def _():
                    wait_store()

                transpose(s)
                store(base + j)

        if odd:
            wait_loads(0 if units_per_w == 1 else (units_per_w - 1) % 2)
            wait_store()
            transpose((units_per_w - 1) % 2)
            store(base + units_per_w - 1)

        wait_store()

        if extra:
            @pl.when(wid < extra)
            def _():
                u = full_units - extra + wid
                fire_loads(u, 0)
                wait_loads(0)
                transpose(0)
                store(u)
                wait_store()

        if tail:
            @pl.when(wid == NW - 1)
            def _():
                pltpu.sync_copy(tail_hbm, tail_v)
                pltpu.sync_copy(tail_v,
                                out_hbm.at[pl.ds(full_units * TU * D,
                                                 tail * D)])

    return transpose_kernel


@functools.lru_cache(maxsize=None)
def _build_gather(R, V, D, NC, NS):
    NW = NC * NS
    rows_per_w = R // NW
    G = 10
    NBUF = 2
    n_chunks = rows_per_w // G
    n_outer = n_chunks // NBUF

    mesh = plsc.VectorSubcoreMesh(core_axis_name="c", subcore_axis_name="s")

    @functools.partial(
        pl.kernel,
        out_type=jax.ShapeDtypeStruct((R, _IDXW, D), jnp.float32),
        mesh=mesh,
        scratch_types=[
            pltpu.VMEM((NBUF, G, _IDXW), jnp.int32),
            pltpu.VMEM((NBUF, G, _IDXW, D), jnp.float32),
            pltpu.SemaphoreType.DMA,
            pltpu.SemaphoreType.DMA,
            pltpu.SemaphoreType.DMA,
        ],
        compiler_params=pltpu.CompilerParams(use_tc_tiling_on_sc=False),
    )
    def gather_kernel(idx_hbm, table_hbm, out_hbm, idx_v, rows_v, sem_idx,
                      sem_gat, sem_st):
        wid = lax.axis_index("s") * NC + lax.axis_index("c")
        base = wid * rows_per_w

        for b in range(NBUF):
            pltpu.async_copy(idx_hbm.at[pl.ds(base + b * G, G)],
                             idx_v.at[b], sem_idx)

        @pl.loop(0, n_outer)
        def _outer(k):
            for b in range(NBUF):
                off = base + (k * NBUF + b) * G

                pltpu.make_async_copy(idx_hbm.at[pl.ds(base, G)],
                                      idx_v.at[b], sem_idx).wait()

                @pl.when(k >= 1)
                def _():
                    pltpu.make_async_copy(rows_v.at[b],
                                          out_hbm.at[pl.ds(base, G)],
                                          sem_st).wait()

                for j in range(G):
                    pltpu.async_copy(table_hbm.at[idx_v.at[b].at[j]],
                                     rows_v.at[b].at[j], sem_gat)

                pltpu.make_async_copy(out_hbm.at[pl.ds(base, G)],
                                      rows_v.at[b], sem_gat).wait()

                @pl.when(k < n_outer - 1)
                def _():
                    pltpu.async_copy(idx_hbm.at[pl.ds(off + NBUF * G, G)],
                                     idx_v.at[b], sem_idx)

                pltpu.async_copy(rows_v.at[b], out_hbm.at[pl.ds(off, G)],
                                 sem_st)

        for b in range(NBUF):
            pltpu.make_async_copy(rows_v.at[b], out_hbm.at[pl.ds(base, G)],
                                  sem_st).wait()

    return gather_kernel


@functools.lru_cache(maxsize=None)
def _build_gather_t(H, B, V, D, NC, NS):
    NW = NC * NS
    BQ = 512
    QN = B // BQ
    R = H * B // _IDXW
    units_total = H * QN
    units_per_w = units_total // NW
    n_outer = units_per_w // 2
    GP = BQ // _IDXW

    mesh = plsc.VectorSubcoreMesh(core_axis_name="c", subcore_axis_name="s")

    @functools.partial(
        pl.kernel,
        out_type=jax.ShapeDtypeStruct((H, D, B), jnp.float32),
        mesh=mesh,
        scratch_types=[
            pltpu.VMEM((2, GP, _IDXW), jnp.int32),
            pltpu.VMEM((2, GP, _IDXW, D), jnp.float32),
            pltpu.VMEM((D, BQ + 1), jnp.float32),
            pltpu.SemaphoreType.DMA,
            pltpu.SemaphoreType.DMA,
            pltpu.SemaphoreType.DMA,
        ],
        compiler_params=pltpu.CompilerParams(use_tc_tiling_on_sc=False,
                                             needs_layout_passes=False),
    )
    def gather_kernel(idx_hbm, table_hbm, out_hbm, idx_v, rows_v, tr_v,
                      sem_i, sem_g, sem_st):
        wid = lax.axis_index("s") * NC + lax.axis_index("c")
        base = wid * units_per_w

        dlo = lax.iota(jnp.int32, 16)
        dhi = dlo + 16

        def fire_gathers(s):
            for g in range(GP):
                pltpu.async_copy(table_hbm.at[idx_v.at[s].at[g]],
                                 rows_v.at[s].at[g], sem_g)

        def wait_gathers(s):
            for g in range(GP):
                pltpu.make_async_copy(table_hbm.at[pl.ds(0, _IDXW)],
                                      rows_v.at[s].at[g], sem_g).wait()

        def out_slice(u):
            return out_hbm.at[u // QN, pl.ds(0, D), pl.ds((u % QN) * BQ, BQ)]

        pltpu.sync_copy(idx_hbm.at[pl.ds(base * GP, GP)], idx_v.at[0])
        fire_gathers(0)
        pltpu.async_copy(idx_hbm.at[pl.ds((base + 1) * GP, GP)],
                         idx_v.at[1], sem_i)

        @pl.loop(0, n_outer)
        def _outer(k):
            for s in range(2):
                j = k * 2 + s
                u = base + j

                wait_gathers(s)

                @pl.when(j < units_per_w - 1)
                def _():
                    pltpu.make_async_copy(idx_hbm.at[pl.ds(0, GP)],
                                          idx_v.at[1 - s], sem_i).wait()
                    fire_gathers(1 - s)

                @pl.when(j < units_per_w - 2)
                def _():
                    pltpu.async_copy(
                        idx_hbm.at[pl.ds((u + 2) * GP, GP)],
                        idx_v.at[s], sem_i)

                @pl.when(j > 0)
                def _():
                    pltpu.make_async_copy(tr_v.at[pl.ds(0, D), pl.ds(0, BQ)],
                                          out_slice(base), sem_st).wait()

                @pl.loop(0, BQ, unroll=8)
                def _tok(t):
                    g = t >> 7
                    c = t & (_IDXW - 1)
                    tv = jnp.full((16,), t, jnp.int32)
                    lo = rows_v[s, g, c, pl.ds(0, 16)]
                    hi = rows_v[s, g, c, pl.ds(16, 16)]
                    plsc.store_scatter(tr_v, [dlo, tv], lo)
                    plsc.store_scatter(tr_v, [dhi, tv], hi)

                pltpu.async_copy(tr_v.at[pl.ds(0, D), pl.ds(0, BQ)],
                                 out_slice(u), sem_st)

        pltpu.make_async_copy(tr_v.at[pl.ds(0, D), pl.ds(0, BQ)],
                              out_slice(base), sem_st).wait()

    return gather_kernel


def kernel(tokens, table):
    B, H = tokens.shape
    V, D = table.shape
    info = plsc.get_sparse_core_info()
    NC, NS = info.num_cores, info.num_subcores

    full = (V // (2 * _IDXW)) * (2 * _IDXW)
    tail_rows = lax.slice(table, (full, 0), (V, D)).reshape(-1)
    t1d = _build_transpose(V, D, NC, NS)(jnp.transpose(table), tail_rows)
    table_lin = t1d.reshape(V, D)

    idx2d = jnp.transpose(tokens).reshape(H * B // _IDXW, _IDXW).astype(jnp.int32)
    out_phys = _build_gather_t(H, B, V, D, NC, NS)(idx2d, table_lin)
    return jnp.transpose(out_phys, (2, 0, 1))


# speedup vs baseline: 1.2150x; 1.2150x over previous
import functools

import jax
import jax.numpy as jnp
from jax import lax
from jax.experimental import pallas as pl
from jax.experimental.pallas import tpu as pltpu
from jax.experimental.pallas import tpu_sc as plsc

_IDXW = 128
_SLABW = 133


@functools.lru_cache(maxsize=None)
def _build_transpose(V, D, NC, NS):
    NW = NC * NS
    RB = D // 8
    TU = 2 * _IDXW
    SW = TU + 5
    full_units = V // TU
    tail = V - full_units * TU
    units_per_w = full_units // NW
    extra = full_units - units_per_w * NW
    n_outer = units_per_w // 2
    odd = units_per_w - 2 * n_outer

    mesh = plsc.VectorSubcoreMesh(core_axis_name="c", subcore_axis_name="s")

    @functools.partial(
        pl.kernel,
        out_type=jax.ShapeDtypeStruct((V * D,), jnp.float32),
        mesh=mesh,
        scratch_types=[
            pltpu.VMEM((2, RB, 8, SW), jnp.float32),
            pltpu.VMEM((TU * D,), jnp.float32),
            pltpu.VMEM((max(tail, 1) * D,), jnp.float32),
            pltpu.SemaphoreType.DMA,
            pltpu.SemaphoreType.DMA,
        ],
        compiler_params=pltpu.CompilerParams(use_tc_tiling_on_sc=True,
                                             needs_layout_passes=False),
    )
    def transpose_kernel(tT_hbm, tail_hbm, out_hbm, slab_v, rows_v, tail_v,
                         sem_in, sem_st):
        wid = lax.axis_index("s") * NC + lax.axis_index("c")
        base = wid * units_per_w

        rv = lax.iota(jnp.int32, 16) & 7
        rb_lo = lax.iota(jnp.int32, 16) >> 3
        rb_hi = rb_lo + 2

        def fire_loads(u, s):
            tok0 = pl.multiple_of(u * TU, _IDXW)
            for rb in range(RB):
                pltpu.async_copy(
                    tT_hbm.at[pl.ds(rb * 8, 8), pl.ds(tok0, TU)],
                    slab_v.at[s, rb, pl.ds(0, 8), pl.ds(0, TU)], sem_in)

        def wait_loads(s):
            for rb in range(RB):
                pltpu.make_async_copy(
                    tT_hbm.at[pl.ds(0, 8), pl.ds(0, TU)],
                    slab_v.at[s, rb, pl.ds(0, 8), pl.ds(0, TU)],
                    sem_in).wait()

        def wait_store():
            pltpu.make_async_copy(rows_v, out_hbm.at[pl.ds(0, TU * D)],
                                  sem_st).wait()

        def transpose(s):
            @plsc.parallel_loop(0, TU, unroll=8)
            def _tok(i):
                cv = jnp.full((16,), i, jnp.int32)
                lo = plsc.load_gather(slab_v.at[s], [rb_lo, rv, cv])
                hi = plsc.load_gather(slab_v.at[s], [rb_hi, rv, cv])
                rows_v[pl.ds(i * D, 16)] = lo
                rows_v[pl.ds(i * D + 16, 16)] = hi

        def store(u):
            pltpu.async_copy(rows_v, out_hbm.at[pl.ds(u * (TU * D), TU * D)],
                             sem_st)

        fire_loads(base, 0)

        @pl.loop(0, n_outer)
        def _outer(k):
            for s in range(2):
                j = k * 2 + s
                wait_loads(s)

                @pl.when(j < units_per_w - 1)
                def _():
                    fire_loads(base + j + 1, 1 - s)

                @pl.when(j > 0)
                def _():
                    wait_store()

                transpose(s)
                store(base + j)

        if odd:
            wait_loads(0 if units_per_w == 1 else (units_per_w - 1) % 2)
            wait_store()
            transpose((units_per_w - 1) % 2)
            store(base + units_per_w - 1)

        wait_store()

        if extra:
            @pl.when(wid < extra)
            def _():
                u = full_units - extra + wid
                fire_loads(u, 0)
                wait_loads(0)
                transpose(0)
                store(u)
                wait_store()

        if tail:
            @pl.when(wid == NW - 1)
            def _():
                pltpu.sync_copy(tail_hbm, tail_v)
                pltpu.sync_copy(tail_v,
                                out_hbm.at[pl.ds(full_units * TU * D,
                                                 tail * D)])

    return transpose_kernel


@functools.lru_cache(maxsize=None)
def _build_gather(R, V, D, NC, NS):
    NW = NC * NS
    rows_per_w = R // NW
    G = 10
    NBUF = 2
    n_chunks = rows_per_w // G
    n_outer = n_chunks // NBUF

    mesh = plsc.VectorSubcoreMesh(core_axis_name="c", subcore_axis_name="s")

    @functools.partial(
        pl.kernel,
        out_type=jax.ShapeDtypeStruct((R, _IDXW, D), jnp.float32),
        mesh=mesh,
        scratch_types=[
            pltpu.VMEM((NBUF, G, _IDXW), jnp.int32),
            pltpu.VMEM((NBUF, G, _IDXW, D), jnp.float32),
            pltpu.SemaphoreType.DMA,
            pltpu.SemaphoreType.DMA,
            pltpu.SemaphoreType.DMA,
        ],
        compiler_params=pltpu.CompilerParams(use_tc_tiling_on_sc=False),
    )
    def gather_kernel(idx_hbm, table_hbm, out_hbm, idx_v, rows_v, sem_idx,
                      sem_gat, sem_st):
        wid = lax.axis_index("s") * NC + lax.axis_index("c")
        base = wid * rows_per_w

        for b in range(NBUF):
            pltpu.async_copy(idx_hbm.at[pl.ds(base + b * G, G)],
                             idx_v.at[b], sem_idx)

        @pl.loop(0, n_outer)
        def _outer(k):
            for b in range(NBUF):
                off = base + (k * NBUF + b) * G

                pltpu.make_async_copy(idx_hbm.at[pl.ds(base, G)],
                                      idx_v.at[b], sem_idx).wait()

                @pl.when(k >= 1)
                def _():
                    pltpu.make_async_copy(rows_v.at[b],
                                          out_hbm.at[pl.ds(base, G)],
                                          sem_st).wait()

                for j in range(G):
                    pltpu.async_copy(table_hbm.at[idx_v.at[b].at[j]],
                                     rows_v.at[b].at[j], sem_gat)

                pltpu.make_async_copy(out_hbm.at[pl.ds(base, G)],
                                      rows_v.at[b], sem_gat).wait()

                @pl.when(k < n_outer - 1)
                def _():
                    pltpu.async_copy(idx_hbm.at[pl.ds(off + NBUF * G, G)],
                                     idx_v.at[b], sem_idx)

                pltpu.async_copy(rows_v.at[b], out_hbm.at[pl.ds(off, G)],
                                 sem_st)

        for b in range(NBUF):
            pltpu.make_async_copy(rows_v.at[b], out_hbm.at[pl.ds(base, G)],
                                  sem_st).wait()

    return gather_kernel


@functools.lru_cache(maxsize=None)
def _build_gather_t(H, B, V, D, NC, NS):
    NW = NC * NS
    BQ = 512
    QN = B // BQ
    R = H * B // _IDXW
    units_total = H * QN
    units_per_w = units_total // NW
    n_outer = units_per_w // 2
    GP = BQ // _IDXW

    mesh = plsc.VectorSubcoreMesh(core_axis_name="c", subcore_axis_name="s")

    @functools.partial(
        pl.kernel,
        out_type=jax.ShapeDtypeStruct((H, D, B), jnp.float32),
        mesh=mesh,
        scratch_types=[
            pltpu.VMEM((2, GP, _IDXW), jnp.int32),
            pltpu.VMEM((2, GP, _IDXW, D), jnp.float32),
            pltpu.VMEM((D, BQ + 1), jnp.float32),
            pltpu.SemaphoreType.DMA,
            pltpu.SemaphoreType.DMA,
            pltpu.SemaphoreType.DMA,
        ],
        compiler_params=pltpu.CompilerParams(use_tc_tiling_on_sc=False,
                                             needs_layout_passes=False),
    )
    def gather_kernel(idx_hbm, table_hbm, out_hbm, idx_v, rows_v, tr_v,
                      sem_i, sem_g, sem_st):
        wid = lax.axis_index("s") * NC + lax.axis_index("c")
        base = wid * units_per_w

        dlo = lax.iota(jnp.int32, 16)
        dhi = dlo + 16

        def fire_gathers(s):
            for g in range(GP):
                pltpu.async_copy(table_hbm.at[idx_v.at[s].at[g]],
                                 rows_v.at[s].at[g], sem_g)

        def wait_gathers(s):
            for g in range(GP):
                pltpu.make_async_copy(table_hbm.at[pl.ds(0, _IDXW)],
                                      rows_v.at[s].at[g], sem_g).wait()

        def out_slice(u):
            return out_hbm.at[u // QN, pl.ds(0, D), pl.ds((u % QN) * BQ, BQ)]

        pltpu.sync_copy(idx_hbm.at[pl.ds(base * GP, GP)], idx_v.at[0])
        fire_gathers(0)
        pltpu.async_copy(idx_hbm.at[pl.ds((base + 1) * GP, GP)],
                         idx_v.at[1], sem_i)

        @pl.loop(0, n_outer)
        def _outer(k):
            for s in range(2):
                j = k * 2 + s
                u = base + j

                wait_gathers(s)

                @pl.when(j < units_per_w - 1)
                def _():
                    pltpu.make_async_copy(idx_hbm.at[pl.ds(0, GP)],
                                          idx_v.at[1 - s], sem_i).wait()
                    fire_gathers(1 - s)

                @pl.when(j < units_per_w - 2)
                def _():
                    pltpu.async_copy(
                        idx_hbm.at[pl.ds((u + 2) * GP, GP)],
                        idx_v.at[s], sem_i)

                @pl.when(j > 0)
                def _():
                    pltpu.make_async_copy(tr_v.at[pl.ds(0, D), pl.ds(0, BQ)],
                                          out_slice(base), sem_st).wait()

                @pl.loop(0, BQ, unroll=8)
                def _tok(t):
                    g = t >> 7
                    c = t & (_IDXW - 1)
                    tv = jnp.full((16,), t, jnp.int32)
                    lo = rows_v[s, g, c, pl.ds(0, 16)]
                    hi = rows_v[s, g, c, pl.ds(16, 16)]
                    plsc.store_scatter(tr_v, [dlo, tv], lo)
                    plsc.store_scatter(tr_v, [dhi, tv], hi)

                pltpu.async_copy(tr_v.at[pl.ds(0, D), pl.ds(0, BQ)],
                                 out_slice(u), sem_st)

        pltpu.make_async_copy(tr_v.at[pl.ds(0, D), pl.ds(0, BQ)],
                              out_slice(base), sem_st).wait()

    return gather_kernel


def kernel(tokens, table):
    B, H = tokens.shape
    V, D = table.shape
    info = plsc.get_sparse_core_info()
    NC, NS = info.num_cores, info.num_subcores

    full = (V // (2 * _IDXW)) * (2 * _IDXW)
    tail_rows = lax.slice(table, (full, 0), (V, D)).reshape(-1)
    t1d = _build_transpose(V, D, NC, NS)(jnp.transpose(table), tail_rows)
    table_lin = t1d.reshape(V, D)

    idx2d = jnp.transpose(tokens).reshape(H * B // _IDXW, _IDXW).astype(jnp.int32)
    out_phys = _build_gather_t(H, B, V, D, NC, NS)(idx2d, table_lin)
    return jnp.transpose(out_phys, (2, 0, 1))


# speedup vs baseline: 1.3068x; 1.0755x over previous
import functools

import jax
import jax.numpy as jnp
from jax import lax
from jax.experimental import pallas as pl
from jax.experimental.pallas import tpu as pltpu
from jax.experimental.pallas import tpu_sc as plsc

_IDXW = 128
_SLABW = 133


@functools.lru_cache(maxsize=None)
def _build_transpose(V, D, NC, NS):
    NW = NC * NS
    RB = D // 8
    TU = 2 * _IDXW
    SW = TU + 5
    full_units = V // TU
    tail = V - full_units * TU
    units_per_w = full_units // NW
    extra = full_units - units_per_w * NW
    n_outer = units_per_w // 2
    odd = units_per_w - 2 * n_outer

    mesh = plsc.VectorSubcoreMesh(core_axis_name="c", subcore_axis_name="s")

    @functools.partial(
        pl.kernel,
        out_type=jax.ShapeDtypeStruct((V * D,), jnp.float32),
        mesh=mesh,
        scratch_types=[
            pltpu.VMEM((2, RB, 8, SW), jnp.float32),
            pltpu.VMEM((TU * D,), jnp.float32),
            pltpu.VMEM((max(tail, 1) * D,), jnp.float32),
            pltpu.SemaphoreType.DMA,
            pltpu.SemaphoreType.DMA,
        ],
        compiler_params=pltpu.CompilerParams(use_tc_tiling_on_sc=True,
                                             needs_layout_passes=False),
    )
    def transpose_kernel(tT_hbm, tail_hbm, out_hbm, slab_v, rows_v, tail_v,
                         sem_in, sem_st):
        wid = lax.axis_index("s") * NC + lax.axis_index("c")
        base = wid * units_per_w

        rv = lax.iota(jnp.int32, 16) & 7
        rb_lo = lax.iota(jnp.int32, 16) >> 3
        rb_hi = rb_lo + 2

        def fire_loads(u, s):
            tok0 = pl.multiple_of(u * TU, _IDXW)
            for rb in range(RB):
                pltpu.async_copy(
                    tT_hbm.at[pl.ds(rb * 8, 8), pl.ds(tok0, TU)],
                    slab_v.at[s, rb, pl.ds(0, 8), pl.ds(0, TU)], sem_in)

        def wait_loads(s):
            for rb in range(RB):
                pltpu.make_async_copy(
                    tT_hbm.at[pl.ds(0, 8), pl.ds(0, TU)],
                    slab_v.at[s, rb, pl.ds(0, 8), pl.ds(0, TU)],
                    sem_in).wait()

        def wait_store():
            pltpu.make_async_copy(rows_v, out_hbm.at[pl.ds(0, TU * D)],
                                  sem_st).wait()

        def transpose(s):
            @plsc.parallel_loop(0, TU, unroll=8)
            def _tok(i):
                cv = jnp.full((16,), i, jnp.int32)
                lo = plsc.load_gather(slab_v.at[s], [rb_lo, rv, cv])
                hi = plsc.load_gather(slab_v.at[s], [rb_hi, rv, cv])
                rows_v[pl.ds(i * D, 16)] = lo
                rows_v[pl.ds(i * D + 16, 16)] = hi

        def store(u):
            pltpu.async_copy(rows_v, out_hbm.at[pl.ds(u * (TU * D), TU * D)],
                             sem_st)

        fire_loads(base, 0)

        @pl.loop(0, n_outer)
        def _outer(k):
            for s in range(2):
                j = k * 2 + s
                wait_loads(s)

                @pl.when(j < units_per_w - 1)
                def _():
                    fire_loads(base + j + 1, 1 - s)

                @pl.when(j > 0)
                def _():
                    wait_store()

                transpose(s)
                store(base + j)

        if odd:
            wait_loads(0 if units_per_w == 1 else (units_per_w - 1) % 2)
            wait_store()
            transpose((units_per_w - 1) % 2)
            store(base + units_per_w - 1)

        wait_store()

        if extra:
            @pl.when(wid < extra)
            def _():
                u = full_units - extra + wid
                fire_loads(u, 0)
                wait_loads(0)
                transpose(0)
                store(u)
                wait_store()

        if tail:
            @pl.when(wid == NW - 1)
            def _():
                pltpu.sync_copy(tail_hbm, tail_v)
                pltpu.sync_copy(tail_v,
                                out_hbm.at[pl.ds(full_units * TU * D,
                                                 tail * D)])

    return transpose_kernel


@functools.lru_cache(maxsize=None)
def _build_gather(R, V, D, NC, NS):
    NW = NC * NS
    rows_per_w = R // NW
    G = 10
    NBUF = 2
    n_chunks = rows_per_w // G
    n_outer = n_chunks // NBUF

    mesh = plsc.VectorSubcoreMesh(core_axis_name="c", subcore_axis_name="s")

    @functools.partial(
        pl.kernel,
        out_type=jax.ShapeDtypeStruct((R, _IDXW, D), jnp.float32),
        mesh=mesh,
        scratch_types=[
            pltpu.VMEM((NBUF, G, _IDXW), jnp.int32),
            pltpu.VMEM((NBUF, G, _IDXW, D), jnp.float32),
            pltpu.SemaphoreType.DMA,
            pltpu.SemaphoreType.DMA,
            pltpu.SemaphoreType.DMA,
        ],
        compiler_params=pltpu.CompilerParams(use_tc_tiling_on_sc=False),
    )
    def gather_kernel(idx_hbm, table_hbm, out_hbm, idx_v, rows_v, sem_idx,
                      sem_gat, sem_st):
        wid = lax.axis_index("s") * NC + lax.axis_index("c")
        base = wid * rows_per_w

        for b in range(NBUF):
            pltpu.async_copy(idx_hbm.at[pl.ds(base + b * G, G)],
                             idx_v.at[b], sem_idx)

        @pl.loop(0, n_outer)
        def _outer(k):
            for b in range(NBUF):
                off = base + (k * NBUF + b) * G

                pltpu.make_async_copy(idx_hbm.at[pl.ds(base, G)],
                                      idx_v.at[b], sem_idx).wait()

                @pl.when(k >= 1)
                def _():
                    pltpu.make_async_copy(rows_v.at[b],
                                          out_hbm.at[pl.ds(base, G)],
                                          sem_st).wait()

                for j in range(G):
                    pltpu.async_copy(table_hbm.at[idx_v.at[b].at[j]],
                                     rows_v.at[b].at[j], sem_gat)

                pltpu.make_async_copy(out_hbm.at[pl.ds(base, G)],
                                      rows_v.at[b], sem_gat).wait()

                @pl.when(k < n_outer - 1)
                def _():
                    pltpu.async_copy(idx_hbm.at[pl.ds(off + NBUF * G, G)],
                                     idx_v.at[b], sem_idx)

                pltpu.async_copy(rows_v.at[b], out_hbm.at[pl.ds(off, G)],
                                 sem_st)

        for b in range(NBUF):
            pltpu.make_async_copy(rows_v.at[b], out_hbm.at[pl.ds(base, G)],
                                  sem_st).wait()

    return gather_kernel


@functools.lru_cache(maxsize=None)
def _build_gather_t(H, B, V, D, NC, NS):
    NW = NC * NS
    BQ = 512
    QN = B // BQ
    R = H * B // _IDXW
    units_total = H * QN
    units_per_w = units_total // NW
    n_outer = units_per_w // 2
    GP = BQ // _IDXW

    mesh = plsc.VectorSubcoreMesh(core_axis_name="c", subcore_axis_name="s")

    @functools.partial(
        pl.kernel,
        out_type=jax.ShapeDtypeStruct((H, D, B), jnp.float32),
        mesh=mesh,
        scratch_types=[
            pltpu.VMEM((2, GP, _IDXW), jnp.int32),
            pltpu.VMEM((2, GP, _IDXW, D), jnp.float32),
            pltpu.VMEM((D, BQ + 1), jnp.float32),
            pltpu.SemaphoreType.DMA,
            pltpu.SemaphoreType.DMA,
            pltpu.SemaphoreType.DMA,
        ],
        compiler_params=pltpu.CompilerParams(use_tc_tiling_on_sc=False,
                                             needs_layout_passes=False),
    )
    def gather_kernel(idx_hbm, table_hbm, out_hbm, idx_v, rows_v, tr_v,
                      sem_i, sem_g, sem_st):
        wid = lax.axis_index("s") * NC + lax.axis_index("c")
        base = wid * units_per_w

        dlo = lax.iota(jnp.int32, 16)
        dhi = dlo + 16

        def fire_gathers(s):
            for g in range(GP):
                pltpu.async_copy(table_hbm.at[idx_v.at[s].at[g]],
                                 rows_v.at[s].at[g], sem_g)

        def wait_gathers(s):
            for g in range(GP):
                pltpu.make_async_copy(table_hbm.at[pl.ds(0, _IDXW)],
                                      rows_v.at[s].at[g], sem_g).wait()

        def out_slice(u):
            return out_hbm.at[u // QN, pl.ds(0, D), pl.ds((u % QN) * BQ, BQ)]

        pltpu.sync_copy(idx_hbm.at[pl.ds(base * GP, GP)], idx_v.at[0])
        fire_gathers(0)
        pltpu.async_copy(idx_hbm.at[pl.ds((base + 1) * GP, GP)],
                         idx_v.at[1], sem_i)

        @pl.loop(0, n_outer)
        def _outer(k):
            for s in range(2):
                j = k * 2 + s
                u = base + j

                wait_gathers(s)

                @pl.when(j < units_per_w - 1)
                def _():
                    pltpu.make_async_copy(idx_hbm.at[pl.ds(0, GP)],
                                          idx_v.at[1 - s], sem_i).wait()
                    fire_gathers(1 - s)

                @pl.when(j < units_per_w - 2)
                def _():
                    pltpu.async_copy(
                        idx_hbm.at[pl.ds((u + 2) * GP, GP)],
                        idx_v.at[s], sem_i)

                @pl.when(j > 0)
                def _():
                    pltpu.make_async_copy(tr_v.at[pl.ds(0, D), pl.ds(0, BQ)],
                                          out_slice(base), sem_st).wait()

                @plsc.parallel_loop(0, BQ, unroll=8)
                def _tok(t):
                    g = t >> 7
                    c = t & (_IDXW - 1)
                    tv = jnp.full((16,), t, jnp.int32)
                    lo = rows_v[s, g, c, pl.ds(0, 16)]
                    hi = rows_v[s, g, c, pl.ds(16, 16)]
                    plsc.store_scatter(tr_v, [dlo, tv], lo)
                    plsc.store_scatter(tr_v, [dhi, tv], hi)

                pltpu.async_copy(tr_v.at[pl.ds(0, D), pl.ds(0, BQ)],
                                 out_slice(u), sem_st)

        pltpu.make_async_copy(tr_v.at[pl.ds(0, D), pl.ds(0, BQ)],
                              out_slice(base), sem_st).wait()

    return gather_kernel


def kernel(tokens, table):
    B, H = tokens.shape
    V, D = table.shape
    info = plsc.get_sparse_core_info()
    NC, NS = info.num_cores, info.num_subcores

    full = (V // (2 * _IDXW)) * (2 * _IDXW)
    tail_rows = lax.slice(table, (full, 0), (V, D)).reshape(-1)
    t1d = _build_transpose(V, D, NC, NS)(jnp.transpose(table), tail_rows)
    table_lin = t1d.reshape(V, D)

    idx2d = jnp.transpose(tokens).reshape(H * B // _IDXW, _IDXW).astype(jnp.int32)
    out_phys = _build_gather_t(H, B, V, D, NC, NS)(idx2d, table_lin)
    return jnp.transpose(out_phys, (2, 0, 1))


# speedup vs baseline: 1.4111x; 1.0798x over previous
import functools

import jax
import jax.numpy as jnp
from jax import lax
from jax.experimental import pallas as pl
from jax.experimental.pallas import tpu as pltpu
from jax.experimental.pallas import tpu_sc as plsc

_IDXW = 128
_SLABW = 133


@functools.lru_cache(maxsize=None)
def _build_transpose(V, D, NC, NS):
    NW = NC * NS
    RB = D // 8
    TU = 2 * _IDXW
    SW = TU + 5
    full_units = V // TU
    tail = V - full_units * TU
    units_per_w = full_units // NW
    extra = full_units - units_per_w * NW
    n_outer = units_per_w // 2
    odd = units_per_w - 2 * n_outer

    mesh = plsc.VectorSubcoreMesh(core_axis_name="c", subcore_axis_name="s")

    @functools.partial(
        pl.kernel,
        out_type=jax.ShapeDtypeStruct((V * D,), jnp.float32),
        mesh=mesh,
        scratch_types=[
            pltpu.VMEM((2, RB, 8, SW), jnp.float32),
            pltpu.VMEM((TU * D,), jnp.float32),
            pltpu.VMEM((max(tail, 1) * D,), jnp.float32),
            pltpu.SemaphoreType.DMA,
            pltpu.SemaphoreType.DMA,
        ],
        compiler_params=pltpu.CompilerParams(use_tc_tiling_on_sc=True,
                                             needs_layout_passes=False),
    )
    def transpose_kernel(tT_hbm, tail_hbm, out_hbm, slab_v, rows_v, tail_v,
                         sem_in, sem_st):
        wid = lax.axis_index("s") * NC + lax.axis_index("c")
        base = wid * units_per_w

        rv = lax.iota(jnp.int32, 16) & 7
        rb_lo = lax.iota(jnp.int32, 16) >> 3
        rb_hi = rb_lo + 2

        def fire_loads(u, s):
            tok0 = pl.multiple_of(u * TU, _IDXW)
            for rb in range(RB):
                pltpu.async_copy(
                    tT_hbm.at[pl.ds(rb * 8, 8), pl.ds(tok0, TU)],
                    slab_v.at[s, rb, pl.ds(0, 8), pl.ds(0, TU)], sem_in)

        def wait_loads(s):
            for rb in range(RB):
                pltpu.make_async_copy(
                    tT_hbm.at[pl.ds(0, 8), pl.ds(0, TU)],
                    slab_v.at[s, rb, pl.ds(0, 8), pl.ds(0, TU)],
                    sem_in).wait()

        def wait_store():
            pltpu.make_async_copy(rows_v, out_hbm.at[pl.ds(0, TU * D)],
                                  sem_st).wait()

        def transpose(s):
            @plsc.parallel_loop(0, TU, unroll=16,
                                carry=jnp.zeros((16,), jnp.int32))
            def _tok(i, cv):
                lo = plsc.load_gather(slab_v.at[s], [rb_lo, rv, cv])
                hi = plsc.load_gather(slab_v.at[s], [rb_hi, rv, cv])
                rows_v[pl.ds(i * D, 16)] = lo
                rows_v[pl.ds(i * D + 16, 16)] = hi
                return cv + 1

        def store(u):
            pltpu.async_copy(rows_v, out_hbm.at[pl.ds(u * (TU * D), TU * D)],
                             sem_st)

        fire_loads(base, 0)

        @pl.loop(0, n_outer)
        def _outer(k):
            for s in range(2):
                j = k * 2 + s
                wait_loads(s)

                @pl.when(j < units_per_w - 1)
                def _():
                    fire_loads(base + j + 1, 1 - s)

                @pl.when(j > 0)
                def _():
                    wait_store()

                transpose(s)
                store(base + j)

        if odd:
            wait_loads(0 if units_per_w == 1 else (units_per_w - 1) % 2)
            wait_store()
            transpose((units_per_w - 1) % 2)
            store(base + units_per_w - 1)

        wait_store()

        if extra:
            @pl.when(wid < extra)
            def _():
                u = full_units - extra + wid
                fire_loads(u, 0)
                wait_loads(0)
                transpose(0)
                store(u)
                wait_store()

        if tail:
            @pl.when(wid == NW - 1)
            def _():
                pltpu.sync_copy(tail_hbm, tail_v)
                pltpu.sync_copy(tail_v,
                                out_hbm.at[pl.ds(full_units * TU * D,
                                                 tail * D)])

    return transpose_kernel


@functools.lru_cache(maxsize=None)
def _build_gather(R, V, D, NC, NS):
    NW = NC * NS
    rows_per_w = R // NW
    G = 10
    NBUF = 2
    n_chunks = rows_per_w // G
    n_outer = n_chunks // NBUF

    mesh = plsc.VectorSubcoreMesh(core_axis_name="c", subcore_axis_name="s")

    @functools.partial(
        pl.kernel,
        out_type=jax.ShapeDtypeStruct((R, _IDXW, D), jnp.float32),
        mesh=mesh,
        scratch_types=[
            pltpu.VMEM((NBUF, G, _IDXW), jnp.int32),
            pltpu.VMEM((NBUF, G, _IDXW, D), jnp.float32),
            pltpu.SemaphoreType.DMA,
            pltpu.SemaphoreType.DMA,
            pltpu.SemaphoreType.DMA,
        ],
        compiler_params=pltpu.CompilerParams(use_tc_tiling_on_sc=False),
    )
    def gather_kernel(idx_hbm, table_hbm, out_hbm, idx_v, rows_v, sem_idx,
                      sem_gat, sem_st):
        wid = lax.axis_index("s") * NC + lax.axis_index("c")
        base = wid * rows_per_w

        for b in range(NBUF):
            pltpu.async_copy(idx_hbm.at[pl.ds(base + b * G, G)],
                             idx_v.at[b], sem_idx)

        @pl.loop(0, n_outer)
        def _outer(k):
            for b in range(NBUF):
                off = base + (k * NBUF + b) * G

                pltpu.make_async_copy(idx_hbm.at[pl.ds(base, G)],
                                      idx_v.at[b], sem_idx).wait()

                @pl.when(k >= 1)
                def _():
                    pltpu.make_async_copy(rows_v.at[b],
                                          out_hbm.at[pl.ds(base, G)],
                                          sem_st).wait()

                for j in range(G):
                    pltpu.async_copy(table_hbm.at[idx_v.at[b].at[j]],
                                     rows_v.at[b].at[j], sem_gat)

                pltpu.make_async_copy(out_hbm.at[pl.ds(base, G)],
                                      rows_v.at[b], sem_gat).wait()

                @pl.when(k < n_outer - 1)
                def _():
                    pltpu.async_copy(idx_hbm.at[pl.ds(off + NBUF * G, G)],
                                     idx_v.at[b], sem_idx)

                pltpu.async_copy(rows_v.at[b], out_hbm.at[pl.ds(off, G)],
                                 sem_st)

        for b in range(NBUF):
            pltpu.make_async_copy(rows_v.at[b], out_hbm.at[pl.ds(base, G)],
                                  sem_st).wait()

    return gather_kernel


@functools.lru_cache(maxsize=None)
def _build_gather_t(H, B, V, D, NC, NS):
    NW = NC * NS
    BQ = 512
    QN = B // BQ
    R = H * B // _IDXW
    units_total = H * QN
    units_per_w = units_total // NW
    n_outer = units_per_w // 2
    GP = BQ // _IDXW

    mesh = plsc.VectorSubcoreMesh(core_axis_name="c", subcore_axis_name="s")

    @functools.partial(
        pl.kernel,
        out_type=jax.ShapeDtypeStruct((H, D, B), jnp.float32),
        mesh=mesh,
        scratch_types=[
            pltpu.VMEM((2, GP, _IDXW), jnp.int32),
            pltpu.VMEM((2, GP, _IDXW, D), jnp.float32),
            pltpu.VMEM((D, BQ + 1), jnp.float32),
            pltpu.SemaphoreType.DMA,
            pltpu.SemaphoreType.DMA,
            pltpu.SemaphoreType.DMA,
        ],
        compiler_params=pltpu.CompilerParams(use_tc_tiling_on_sc=False,
                                             needs_layout_passes=False),
    )
    def gather_kernel(idx_hbm, table_hbm, out_hbm, idx_v, rows_v, tr_v,
                      sem_i, sem_g, sem_st):
        wid = lax.axis_index("s") * NC + lax.axis_index("c")
        base = wid * units_per_w

        dlo = lax.iota(jnp.int32, 16)
        dhi = dlo + 16

        def fire_gathers(s):
            for g in range(GP):
                pltpu.async_copy(table_hbm.at[idx_v.at[s].at[g]],
                                 rows_v.at[s].at[g], sem_g)

        def wait_gathers(s):
            for g in range(GP):
                pltpu.make_async_copy(table_hbm.at[pl.ds(0, _IDXW)],
                                      rows_v.at[s].at[g], sem_g).wait()

        def out_slice(u):
            return out_hbm.at[u // QN, pl.ds(0, D), pl.ds((u % QN) * BQ, BQ)]

        pltpu.sync_copy(idx_hbm.at[pl.ds(base * GP, GP)], idx_v.at[0])
        fire_gathers(0)
        pltpu.async_copy(idx_hbm.at[pl.ds((base + 1) * GP, GP)],
                         idx_v.at[1], sem_i)

        @pl.loop(0, n_outer)
        def _outer(k):
            for s in range(2):
                j = k * 2 + s
                u = base + j

                wait_gathers(s)

                @pl.when(j < units_per_w - 1)
                def _():
                    pltpu.make_async_copy(idx_hbm.at[pl.ds(0, GP)],
                                          idx_v.at[1 - s], sem_i).wait()
                    fire_gathers(1 - s)

                @pl.when(j < units_per_w - 2)
                def _():
                    pltpu.async_copy(
                        idx_hbm.at[pl.ds((u + 2) * GP, GP)],
                        idx_v.at[s], sem_i)

                @pl.when(j > 0)
                def _():
                    pltpu.make_async_copy(tr_v.at[pl.ds(0, D), pl.ds(0, BQ)],
                                          out_slice(base), sem_st).wait()

                @plsc.parallel_loop(0, BQ, unroll=8)
                def _tok(t):
                    g = t >> 7
                    c = t & (_IDXW - 1)
                    tv = jnp.full((16,), t, jnp.int32)
                    lo = rows_v[s, g, c, pl.ds(0, 16)]
                    hi = rows_v[s, g, c, pl.ds(16, 16)]
                    plsc.store_scatter(tr_v, [dlo, tv], lo)
                    plsc.store_scatter(tr_v, [dhi, tv], hi)

                pltpu.async_copy(tr_v.at[pl.ds(0, D), pl.ds(0, BQ)],
                                 out_slice(u), sem_st)

        pltpu.make_async_copy(tr_v.at[pl.ds(0, D), pl.ds(0, BQ)],
                              out_slice(base), sem_st).wait()

    return gather_kernel


def kernel(tokens, table):
    B, H = tokens.shape
    V, D = table.shape
    info = plsc.get_sparse_core_info()
    NC, NS = info.num_cores, info.num_subcores

    full = (V // (2 * _IDXW)) * (2 * _IDXW)
    tail_rows = lax.slice(table, (full, 0), (V, D)).reshape(-1)
    t1d = _build_transpose(V, D, NC, NS)(jnp.transpose(table), tail_rows)
    table_lin = t1d.reshape(V, D)

    idx2d = jnp.transpose(tokens).reshape(H * B // _IDXW, _IDXW).astype(jnp.int32)
    out_phys = _build_gather_t(H, B, V, D, NC, NS)(idx2d, table_lin)
    return jnp.transpose(out_phys, (2, 0, 1))
